# baseline (speedup 1.0000x reference)
import functools

import jax
import jax.numpy as jnp
from jax import lax
from jax.experimental import pallas as pl
from jax.experimental.pallas import tpu as pltpu
from jax.experimental.pallas import tpu_sc as plsc


def _build_sc_gather(B, D, M, A):
    info = plsc.get_sparse_core_info()
    NC, NS = info.num_cores, info.num_subcores
    NW = NC * NS
    CHUNK = B // NW
    NGRP = CHUNK // 128
    NV = 128 // 16
    pw = [pow(31, i, M) for i in range(D)]

    mesh = plsc.VectorSubcoreMesh(core_axis_name="c", subcore_axis_name="s")

    @functools.partial(
        pl.kernel,
        out_type=[
            jax.ShapeDtypeStruct((B, A), jnp.float32),
            jax.ShapeDtypeStruct((B, A), jnp.float32),
        ],
        mesh=mesh,
        compiler_params=pltpu.CompilerParams(use_tc_tiling_on_sc=False),
        scratch_types=[
            pltpu.VMEM((D, CHUNK), jnp.int32),
            pltpu.VMEM((NGRP, 128), jnp.int32),
            pltpu.VMEM((CHUNK, A), jnp.float32),
            pltpu.VMEM((CHUNK, A), jnp.float32),
            pltpu.SemaphoreType.DMA,
        ],
    )
    def sc_gather(xt_hbm, q_hbm, p_hbm, qout_hbm, pout_hbm, xv, idxv, qv, pv, sem):
        wid = lax.axis_index("s") * NC + lax.axis_index("c")
        base = wid * CHUNK
        pltpu.sync_copy(xt_hbm.at[:, pl.ds(base, CHUNK)], xv)
        for j in range(NGRP):
            for k in range(NV):
                c0 = j * 128 + k * 16
                acc = xv[0, pl.ds(c0, 16)] * pw[0]
                for d in range(1, D):
                    acc = acc + xv[d, pl.ds(c0, 16)] * pw[d]
                idxv[j, pl.ds(k * 16, 16)] = lax.rem(acc, M)
        copies = []
        for j in range(NGRP):
            copies.append(pltpu.async_copy(
                q_hbm.at[idxv.at[j]], qv.at[pl.ds(j * 128, 128)], sem))
            copies.append(pltpu.async_copy(
                p_hbm.at[idxv.at[j]], pv.at[pl.ds(j * 128, 128)], sem))
        for c in copies:
            c.wait()
        pltpu.sync_copy(qv, qout_hbm.at[pl.ds(base, CHUNK)])
        pltpu.sync_copy(pv, pout_hbm.at[pl.ds(base, CHUNK)])

    return sc_gather


def _dense_body(B, A, q_ref, a_ref, vals_ref, vidx_ref, probs_ref, ent_ref):
    q = q_ref[...]
    a = a_ref[...]
    rowmax = jnp.max(q, axis=1)
    vals_ref[...] = rowmax
    col = lax.broadcasted_iota(jnp.int32, q.shape, 1)
    vidx_ref[...] = jnp.min(jnp.where(q == rowmax[:, None], col, A), axis=1)
    am = jnp.max(a, axis=1, keepdims=True)
    s = a - am
    e = jnp.exp(s)
    z = jnp.sum(e, axis=1, keepdims=True)
    p = e / z
    probs_ref[...] = p
    lp = s - jnp.log(z)
    ent_blk = -jnp.sum(lp * p)

    @pl.when(pl.program_id(0) == 0)
    def _():
        ent_ref[...] = jnp.zeros((1, 1), jnp.float32)

    ent_ref[...] += jnp.full((1, 1), ent_blk / B, jnp.float32)


def _dense(qvals, aprobs):
    B, A = qvals.shape
    BLK = 1024
    grid = (B // BLK,)
    return pl.pallas_call(
        functools.partial(_dense_body, B, A),
        grid=grid,
        in_specs=[
            pl.BlockSpec((BLK, A), lambda i: (i, 0)),
            pl.BlockSpec((BLK, A), lambda i: (i, 0)),
        ],
        out_specs=[
            pl.BlockSpec((BLK,), lambda i: (i,)),
            pl.BlockSpec((BLK,), lambda i: (i,)),
            pl.BlockSpec((BLK, A), lambda i: (i, 0)),
            pl.BlockSpec((1, 1), lambda i: (0, 0)),
        ],
        out_shape=[
            jax.ShapeDtypeStruct((B,), jnp.float32),
            jax.ShapeDtypeStruct((B,), jnp.int32),
            jax.ShapeDtypeStruct((B, A), jnp.float32),
            jax.ShapeDtypeStruct((1, 1), jnp.float32),
        ],
    )(qvals, aprobs)


def kernel(x, Qtable, aprob_table):
    B, D = x.shape
    M, A = Qtable.shape
    sc_gather = _build_sc_gather(B, D, M, A)
    qvals, aprobs = sc_gather(x.T, Qtable, aprob_table)
    values, vidx, probs, ent = _dense(qvals, aprobs)
    return (values, vidx, ent.reshape(()), probs, qvals)
